# baseline (speedup 1.0000x reference)
import functools

import jax
import jax.numpy as jnp
from jax import lax
from jax.experimental import pallas as pl
from jax.experimental.pallas import tpu as pltpu
from jax.experimental.pallas import tpu_sc as plsc

N = 2048
D = 1024
E = 8
K = 2
T = 256
NSLOT = 24
NROWS = NSLOT * T
SEG_SZ = N + 16
SCAT_ROWS = 17
SCAT_SZ = SCAT_ROWS * 128
CHUNKS = N // 16


def _router_body(x_ref, gwt_ref, idx_ref, w_ref):
    x = x_ref[...]
    lg = jnp.dot(x, gwt_ref[...], preferred_element_type=jnp.float32)
    m = jnp.max(lg, axis=-1, keepdims=True)
    ex = jnp.exp(lg - m)
    s = ex / jnp.sum(ex, axis=-1, keepdims=True)
    m1 = s[:, 0:1]
    i1 = jnp.zeros_like(m1, dtype=jnp.int32)
    m2 = jnp.full_like(m1, -1.0)
    i2 = jnp.full_like(i1, -1)
    for e in range(1, E):
        v = s[:, e : e + 1]
        gt1 = v > m1
        gt2 = v > m2
        m2n = jnp.where(gt1, m1, jnp.where(gt2, v, m2))
        i2n = jnp.where(gt1, i1, jnp.where(gt2, e, i2))
        m1 = jnp.where(gt1, v, m1)
        i1 = jnp.where(gt1, e, i1)
        m2, i2 = m2n, i2n
    idx_ref[:, 0:1] = i1
    idx_ref[:, 1:2] = i2
    idx_ref[:, 2:8] = jnp.zeros((i1.shape[0], 6), jnp.int32)
    w_ref[:, 0:1] = m1
    w_ref[:, 1:2] = m2
    w_ref[:, 2:8] = jnp.zeros((i1.shape[0], 6), jnp.float32)


def _meta_body(idx0h, idx1h, sorted_tok, p0, p1, slots, idxbuf0, idxbuf1, seg,
               tok0f, pos0f, tok1f, pos1f, tok0r, pos0r, tok1r, pos1r, zbuf,
               slotsv, sh0, sh1, sh2, sh3, sem):
    sid = lax.axis_index("s")
    iota = lax.iota(jnp.int32, 16)
    zeros16 = jnp.zeros((16,), jnp.int32)

    @pl.when(sid >= E)
    def _zero_fill():
        for j in range(16):
            zbuf[pl.ds(j * 16, 16)] = zeros16
        share = NROWS // E
        base = (sid - E) * share
        for j in range(share // 256):
            pltpu.sync_copy(
                zbuf, sorted_tok.at[pl.ds(pl.multiple_of(base + j * 256, 256), 256)])

    pltpu.sync_copy(idx0h, idxbuf0)
    pltpu.sync_copy(idx1h, idxbuf1)

    def c_body(c, accs):
        i0 = idxbuf0[pl.ds(c * 16, 16)]
        i1 = idxbuf1[pl.ds(c * 16, 16)]
        return tuple(
            accs[e]
            + jnp.where(i0 == e, 1, 0)
            + jnp.where(i1 == e, 1, 0)
            for e in range(E)
        )

    accs = lax.fori_loop(0, CHUNKS, c_body, tuple(zeros16 for _ in range(E)))
    cnts = [jnp.sum(accs[e]) for e in range(E)]
    nts = [(cnts[e] + (T - 1)) // T for e in range(E)]
    seg_base = jnp.int32(0)
    ntiles_mine = jnp.int32(0)
    total_slots = jnp.int32(0)
    for e in range(E):
        seg_base = seg_base + jnp.where(sid > e, nts[e], 0) * T
        ntiles_mine = ntiles_mine + jnp.where(sid == e, nts[e], 0)
        total_slots = total_slots + nts[e]

    for j in range(SEG_SZ // 16):
        seg[pl.ds(j * 16, 16)] = zeros16
    dummy = jnp.full((16,), N, jnp.int32) + sid
    for j in range(SCAT_SZ // 16):
        tok0f[pl.ds(j * 16, 16)] = dummy
        tok1f[pl.ds(j * 16, 16)] = dummy

    def p_body(c, offs):
        off, off0, off1 = offs
        tok = c * 16 + iota
        i0 = idxbuf0[pl.ds(c * 16, 16)]
        i1 = idxbuf1[pl.ds(c * 16, 16)]
        m0 = i0 == sid
        c0v = plsc.cumsum(jnp.where(m0, 1, 0))
        c0 = jnp.max(c0v)
        plsc.store_compressed(seg.at[pl.ds(off, 16)], tok, mask=m0)
        plsc.store_compressed(tok0f.at[pl.ds(off0, 16)], tok, mask=m0)
        plsc.store_compressed(
            pos0f.at[pl.ds(off0, 16)], seg_base + off + c0v - 1, mask=m0)
        m1 = i1 == sid
        c1v = plsc.cumsum(jnp.where(m1, 1, 0))
        c1 = jnp.max(c1v)
        plsc.store_compressed(seg.at[pl.ds(off + c0, 16)], tok, mask=m1)
        plsc.store_compressed(tok1f.at[pl.ds(off1, 16)], tok, mask=m1)
        plsc.store_compressed(
            pos1f.at[pl.ds(off1, 16)], seg_base + off + c0 + c1v - 1, mask=m1)
        return off + c0 + c1, off0 + c0, off1 + c1

    lax.fori_loop(0, CHUNKS, p_body, (jnp.int32(0), jnp.int32(0), jnp.int32(0)))

    pltpu.sync_copy(tok0f, sh0.at[sid])
    pltpu.sync_copy(pos0f, sh1.at[sid])
    pltpu.sync_copy(tok1f, sh2.at[sid])
    pltpu.sync_copy(pos1f, sh3.at[sid])
    for j in range(SCAT_ROWS):
        pltpu.sync_copy(sh0.at[sid, pl.ds(j * 128, 128)], tok0r.at[j])
        pltpu.sync_copy(sh1.at[sid, pl.ds(j * 128, 128)], pos0r.at[j])
        pltpu.sync_copy(sh2.at[sid, pl.ds(j * 128, 128)], tok1r.at[j])
        pltpu.sync_copy(sh3.at[sid, pl.ds(j * 128, 128)], pos1r.at[j])

    @pl.when(sid == 0)
    def _slots():
        for c in range(2):
            p = c * 16 + iota
            v = jnp.zeros((16,), jnp.int32)
            sb = jnp.int32(0)
            for e in range(E):
                v = v + jnp.where((p >= sb) & (p < sb + nts[e]), e, 0)
                sb = sb + nts[e]
            v = v + jnp.where(p == NSLOT, total_slots, 0)
            slotsv[pl.ds(c * 16, 16)] = v

    plsc.subcore_barrier()

    def d_body(j, _):
        pltpu.sync_copy(
            seg.at[pl.ds(pl.multiple_of(j * T, T), T)],
            sorted_tok.at[pl.ds(pl.multiple_of(seg_base + j * T, T), T)])
        return 0

    lax.fori_loop(0, ntiles_mine, d_body, 0)
    for j in range(SCAT_ROWS):
        pltpu.async_copy(pos0r.at[j], p0.at[tok0r.at[j]], sem).wait()
        pltpu.async_copy(pos1r.at[j], p1.at[tok1r.at[j]], sem).wait()

    @pl.when(sid == 0)
    def _wslots():
        pltpu.sync_copy(slotsv, slots)


def _xgather_body(x_hbm, st_hbm, xs_hbm, idxv, rows, sem):
    wid = lax.axis_index("s") * 2 + lax.axis_index("c")
    for j in range(3):
        base = pl.multiple_of(wid * 192 + j * 64, 64)
        pltpu.sync_copy(st_hbm.at[pl.ds(base, 64)], idxv)
        pltpu.async_copy(x_hbm.at[idxv], rows, sem).wait()
        pltpu.sync_copy(rows, xs_hbm.at[pl.ds(base, 64)])


def _ogather_body(ob_hbm, p0_hbm, p1_hbm, b0_hbm, b1_hbm, idxv, rows, sem):
    wid = lax.axis_index("s") * 2 + lax.axis_index("c")
    base = pl.multiple_of(wid * 64, 64)
    pltpu.sync_copy(p0_hbm.at[pl.ds(base, 64)], idxv)
    pltpu.async_copy(ob_hbm.at[idxv], rows, sem).wait()
    pltpu.sync_copy(rows, b0_hbm.at[pl.ds(base, 64)])
    pltpu.sync_copy(p1_hbm.at[pl.ds(base, 64)], idxv)
    pltpu.async_copy(ob_hbm.at[idxv], rows, sem).wait()
    pltpu.sync_copy(rows, b1_hbm.at[pl.ds(base, 64)])


def _expert_body(slots_ref, x_ref, wg_ref, wu_ref, wd_ref, out_ref):
    t = pl.program_id(0)

    @pl.when(t < slots_ref[NSLOT])
    def _go():
        x = x_ref[...].astype(jnp.bfloat16)
        xg = jnp.dot(x, wg_ref[0], preferred_element_type=jnp.float32)
        xu = jnp.dot(x, wu_ref[0], preferred_element_type=jnp.float32)
        h = (xg * jax.nn.sigmoid(xg) * xu).astype(jnp.bfloat16)
        out_ref[...] = jnp.dot(h, wd_ref[0], preferred_element_type=jnp.float32)


def _combine_body(x_ref, b0_ref, b1_ref, w_ref, swg_ref, swu_ref, swd_ref, y_ref):
    x = x_ref[...]
    xg = jnp.dot(x, swg_ref[...], preferred_element_type=jnp.float32)
    xu = jnp.dot(x, swu_ref[...], preferred_element_type=jnp.float32)
    h = (xg * jax.nn.sigmoid(xg) * xu).astype(jnp.bfloat16)
    acc = jnp.dot(h, swd_ref[...], preferred_element_type=jnp.float32)
    acc = acc + b0_ref[...] * w_ref[:, 0:1] + b1_ref[...] * w_ref[:, 1:2]
    y_ref[...] = acc


def kernel(hidden_states, gate_weight, w_gate, w_up, w_down, sw_gate, sw_up, sw_down):
    Bsz, S, _ = hidden_states.shape
    F = w_gate.shape[2]
    FS = sw_gate.shape[1]
    nt = N // T

    x = hidden_states.reshape(N, D)
    x16 = x.astype(jnp.bfloat16)
    gwt = gate_weight.T
    wg16 = w_gate.astype(jnp.bfloat16)
    wu16 = w_up.astype(jnp.bfloat16)
    wd16 = w_down.astype(jnp.bfloat16)
    swg16 = sw_gate.astype(jnp.bfloat16)
    swu16 = sw_up.astype(jnp.bfloat16)
    swd16 = sw_down.astype(jnp.bfloat16)

    idx2d, w2d = pl.pallas_call(
        _router_body,
        grid=(nt,),
        in_specs=[
            pl.BlockSpec((T, D), lambda t: (t, 0)),
            pl.BlockSpec((D, E), lambda t: (0, 0)),
        ],
        out_specs=[
            pl.BlockSpec((T, E), lambda t: (t, 0)),
            pl.BlockSpec((T, E), lambda t: (t, 0)),
        ],
        out_shape=[
            jax.ShapeDtypeStruct((N, E), jnp.int32),
            jax.ShapeDtypeStruct((N, E), jnp.float32),
        ],
    )(x, gwt)

    meta = pl.kernel(
        _meta_body,
        out_type=[
            jax.ShapeDtypeStruct((NROWS,), jnp.int32),
            jax.ShapeDtypeStruct((SCAT_SZ,), jnp.int32),
            jax.ShapeDtypeStruct((SCAT_SZ,), jnp.int32),
            jax.ShapeDtypeStruct((32,), jnp.int32),
        ],
        mesh=plsc.VectorSubcoreMesh(
            core_axis_name="c", subcore_axis_name="s", num_cores=1),
        compiler_params=pltpu.CompilerParams(needs_layout_passes=False),
        scratch_types=[
            pltpu.VMEM((N,), jnp.int32),
            pltpu.VMEM((N,), jnp.int32),
            pltpu.VMEM((SEG_SZ,), jnp.int32),
            pltpu.VMEM((SCAT_SZ,), jnp.int32),
            pltpu.VMEM((SCAT_SZ,), jnp.int32),
            pltpu.VMEM((SCAT_SZ,), jnp.int32),
            pltpu.VMEM((SCAT_SZ,), jnp.int32),
            pltpu.VMEM((SCAT_ROWS, 128), jnp.int32),
            pltpu.VMEM((SCAT_ROWS, 128), jnp.int32),
            pltpu.VMEM((SCAT_ROWS, 128), jnp.int32),
            pltpu.VMEM((SCAT_ROWS, 128), jnp.int32),
            pltpu.VMEM((256,), jnp.int32),
            pltpu.VMEM((32,), jnp.int32),
            pltpu.VMEM_SHARED((16, SCAT_SZ), jnp.int32),
            pltpu.VMEM_SHARED((16, SCAT_SZ), jnp.int32),
            pltpu.VMEM_SHARED((16, SCAT_SZ), jnp.int32),
            pltpu.VMEM_SHARED((16, SCAT_SZ), jnp.int32),
            pltpu.SemaphoreType.DMA,
        ],
    )(idx2d[:, 0], idx2d[:, 1])
    sorted_tok, p0, p1, slots = meta

    x_sorted = pl.kernel(
        _xgather_body,
        out_type=jax.ShapeDtypeStruct((NROWS, D), jnp.float32),
        mesh=plsc.VectorSubcoreMesh(core_axis_name="c", subcore_axis_name="s"),
        scratch_types=[
            pltpu.VMEM((64,), jnp.int32),
            pltpu.VMEM((64, D), jnp.float32),
            pltpu.SemaphoreType.DMA,
        ],
    )(x, sorted_tok)

    out_buf = pl.pallas_call(
        _expert_body,
        grid_spec=pltpu.PrefetchScalarGridSpec(
            num_scalar_prefetch=1,
            grid=(NSLOT,),
            in_specs=[
                pl.BlockSpec((T, D), lambda t, m: (t, 0)),
                pl.BlockSpec((1, D, F), lambda t, m: (m[t], 0, 0)),
                pl.BlockSpec((1, D, F), lambda t, m: (m[t], 0, 0)),
                pl.BlockSpec((1, F, D), lambda t, m: (m[t], 0, 0)),
            ],
            out_specs=pl.BlockSpec((T, D), lambda t, m: (t, 0)),
        ),
        out_shape=jax.ShapeDtypeStruct((NROWS, D), jnp.float32),
    )(slots, x_sorted, wg16, wu16, wd16)

    buf0, buf1 = pl.kernel(
        _ogather_body,
        out_type=[
            jax.ShapeDtypeStruct((N, D), jnp.float32),
            jax.ShapeDtypeStruct((N, D), jnp.float32),
        ],
        mesh=plsc.VectorSubcoreMesh(core_axis_name="c", subcore_axis_name="s"),
        scratch_types=[
            pltpu.VMEM((64,), jnp.int32),
            pltpu.VMEM((64, D), jnp.float32),
            pltpu.SemaphoreType.DMA,
        ],
    )(out_buf, p0, p1)

    y = pl.pallas_call(
        _combine_body,
        grid=(nt,),
        in_specs=[
            pl.BlockSpec((T, D), lambda t: (t, 0)),
            pl.BlockSpec((T, D), lambda t: (t, 0)),
            pl.BlockSpec((T, D), lambda t: (t, 0)),
            pl.BlockSpec((T, E), lambda t: (t, 0)),
            pl.BlockSpec((D, FS), lambda t: (0, 0)),
            pl.BlockSpec((D, FS), lambda t: (0, 0)),
            pl.BlockSpec((FS, D), lambda t: (0, 0)),
        ],
        out_specs=pl.BlockSpec((T, D), lambda t: (t, 0)),
        out_shape=jax.ShapeDtypeStruct((N, D), jnp.float32),
    )(x16, buf0, buf1, w2d, swg16, swu16, swd16)

    return y.reshape(Bsz, S, D)


# speedup vs baseline: 8.5023x; 8.5023x over previous
import jax
import jax.numpy as jnp
from jax import lax
from jax.experimental import pallas as pl
from jax.experimental.pallas import tpu as pltpu
from jax.experimental.pallas import tpu_sc as plsc

N = 2048
D = 1024
E = 8
T = 256
NSLOT = 24
NROWS = NSLOT * T
SEG_SZ = N + 16
NPAIR_PAD = 2 * N + 32
CHUNKS = N // 16


def _router_body(x_ref, gwt_ref, idx_ref, w_ref):
    x = x_ref[...]
    lg = jnp.dot(x, gwt_ref[...], preferred_element_type=jnp.float32)
    m = jnp.max(lg, axis=-1, keepdims=True)
    ex = jnp.exp(lg - m)
    s = ex / jnp.sum(ex, axis=-1, keepdims=True)
    m1 = s[:, 0:1]
    i1 = jnp.zeros_like(m1, dtype=jnp.int32)
    m2 = jnp.full_like(m1, -1.0)
    i2 = jnp.full_like(i1, -1)
    for e in range(1, E):
        v = s[:, e : e + 1]
        gt1 = v > m1
        gt2 = v > m2
        m2n = jnp.where(gt1, m1, jnp.where(gt2, v, m2))
        i2n = jnp.where(gt1, i1, jnp.where(gt2, e, i2))
        m1 = jnp.where(gt1, v, m1)
        i1 = jnp.where(gt1, e, i1)
        m2, i2 = m2n, i2n
    idx_ref[:, 0:1] = i1
    idx_ref[:, 1:2] = i2
    idx_ref[:, 2:8] = jnp.zeros((i1.shape[0], 6), jnp.int32)
    w_ref[:, 0:1] = m1
    w_ref[:, 1:2] = m2
    w_ref[:, 2:8] = jnp.zeros((i1.shape[0], 6), jnp.float32)


def _meta_body(idx0h, idx1h, sorted_tok, pairdst, slots, idxbuf0, idxbuf1,
               seg, segp, zbuf, slotsv):
    sid = lax.axis_index("s")
    iota = lax.iota(jnp.int32, 16)
    zeros16 = jnp.zeros((16,), jnp.int32)
    dummy = jnp.full((16,), 2 * N, jnp.int32) + sid

    @pl.when(sid >= E)
    def _zero_fill():
        share = NROWS // E
        base = (sid - E) * share

        def zf(j, _):
            zbuf[pl.ds(pl.multiple_of(j * 16, 16), 16)] = zeros16
            return 0

        lax.fori_loop(0, 16, zf, 0)
        for j in range(share // 256):
            pltpu.sync_copy(
                zbuf, sorted_tok.at[pl.ds(pl.multiple_of(base + j * 256, 256), 256)])

        def zf2(j, _):
            zbuf[pl.ds(pl.multiple_of(j * 16, 16), 16)] = dummy
            return 0

        lax.fori_loop(0, 16, zf2, 0)
        for j in range(share // 256):
            pltpu.sync_copy(
                zbuf, pairdst.at[pl.ds(pl.multiple_of(base + j * 256, 256), 256)])

    pltpu.sync_copy(idx0h, idxbuf0)
    pltpu.sync_copy(idx1h, idxbuf1)

    def c_body(c, accs):
        i0 = idxbuf0[pl.ds(c * 16, 16)]
        i1 = idxbuf1[pl.ds(c * 16, 16)]
        return tuple(
            accs[e]
            + jnp.where(i0 == e, 1, 0)
            + jnp.where(i1 == e, 1, 0)
            for e in range(E)
        )

    accs = lax.fori_loop(0, CHUNKS, c_body, tuple(zeros16 for _ in range(E)))
    cnts = [jnp.sum(accs[e]) for e in range(E)]
    nts = [(cnts[e] + (T - 1)) // T for e in range(E)]
    seg_base = jnp.int32(0)
    ntiles_mine = jnp.int32(0)
    total_slots = jnp.int32(0)
    for e in range(E):
        seg_base = seg_base + jnp.where(sid > e, nts[e], 0) * T
        ntiles_mine = ntiles_mine + jnp.where(sid == e, nts[e], 0)
        total_slots = total_slots + nts[e]

    def i_body(j, _):
        seg[pl.ds(pl.multiple_of(j * 16, 16), 16)] = zeros16
        segp[pl.ds(pl.multiple_of(j * 16, 16), 16)] = dummy
        return 0

    lax.fori_loop(0, SEG_SZ // 16, i_body, 0)

    def p_body(c, off):
        tok = c * 16 + iota
        i0 = idxbuf0[pl.ds(c * 16, 16)]
        i1 = idxbuf1[pl.ds(c * 16, 16)]
        m0 = i0 == sid
        c0 = jnp.sum(jnp.where(m0, 1, 0))
        plsc.store_compressed(seg.at[pl.ds(off, 16)], tok, mask=m0)
        plsc.store_compressed(segp.at[pl.ds(off, 16)], tok, mask=m0)
        m1 = i1 == sid
        c1 = jnp.sum(jnp.where(m1, 1, 0))
        plsc.store_compressed(seg.at[pl.ds(off + c0, 16)], tok, mask=m1)
        plsc.store_compressed(segp.at[pl.ds(off + c0, 16)], tok + N, mask=m1)
        return off + c0 + c1

    lax.fori_loop(0, CHUNKS, p_body, jnp.int32(0))

    @pl.when(sid == 0)
    def _slots():
        for c in range(2):
            p = c * 16 + iota
            v = jnp.zeros((16,), jnp.int32)
            sb = jnp.int32(0)
            for e in range(E):
                v = v + jnp.where((p >= sb) & (p < sb + nts[e]), e, 0)
                sb = sb + nts[e]
            v = v + jnp.where(p == NSLOT, total_slots, 0)
            slotsv[pl.ds(c * 16, 16)] = v

    plsc.subcore_barrier()

    def d_body(j, _):
        src = pl.ds(pl.multiple_of(j * T, T), T)
        dst = pl.ds(pl.multiple_of(seg_base + j * T, T), T)
        pltpu.sync_copy(seg.at[src], sorted_tok.at[dst])
        pltpu.sync_copy(segp.at[src], pairdst.at[dst])
        return 0

    lax.fori_loop(0, ntiles_mine, d_body, 0)

    @pl.when(sid == 0)
    def _wslots():
        pltpu.sync_copy(slotsv, slots)


def _xgather_body(x_hbm, st_hbm, xs_hbm, idxv, rows, sem):
    wid = lax.axis_index("s") * 2 + lax.axis_index("c")
    for j in range(3):
        base = pl.multiple_of(wid * 192 + j * 64, 64)
        pltpu.sync_copy(st_hbm.at[pl.ds(base, 64)], idxv)
        pltpu.async_copy(x_hbm.at[idxv], rows, sem).wait()
        pltpu.sync_copy(rows, xs_hbm.at[pl.ds(base, 64)])


def _oscatter_body(ob_hbm, pd_hbm, yp_hbm, idxv, rows, sem):
    wid = lax.axis_index("s") * 2 + lax.axis_index("c")
    for j in range(3):
        base = pl.multiple_of(wid * 192 + j * 64, 64)
        pltpu.sync_copy(pd_hbm.at[pl.ds(base, 64)], idxv)
        pltpu.sync_copy(ob_hbm.at[pl.ds(base, 64)], rows)
        pltpu.async_copy(rows, yp_hbm.at[idxv], sem).wait()


def _expert_body(slots_ref, x_ref, wg_ref, wu_ref, wd_ref, out_ref):
    t = pl.program_id(0)

    @pl.when(t < slots_ref[NSLOT])
    def _go():
        x = x_ref[...].astype(jnp.bfloat16)
        xg = jnp.dot(x, wg_ref[0], preferred_element_type=jnp.float32)
        xu = jnp.dot(x, wu_ref[0], preferred_element_type=jnp.float32)
        h = (xg * jax.nn.sigmoid(xg) * xu).astype(jnp.bfloat16)
        out_ref[...] = jnp.dot(h, wd_ref[0], preferred_element_type=jnp.float32)


def _combine_body(x_ref, b0_ref, b1_ref, w_ref, swg_ref, swu_ref, swd_ref, y_ref):
    x = x_ref[...]
    xg = jnp.dot(x, swg_ref[...], preferred_element_type=jnp.float32)
    xu = jnp.dot(x, swu_ref[...], preferred_element_type=jnp.float32)
    h = (xg * jax.nn.sigmoid(xg) * xu).astype(jnp.bfloat16)
    acc = jnp.dot(h, swd_ref[...], preferred_element_type=jnp.float32)
    acc = acc + b0_ref[...] * w_ref[:, 0:1] + b1_ref[...] * w_ref[:, 1:2]
    y_ref[...] = acc


def kernel(hidden_states, gate_weight, w_gate, w_up, w_down, sw_gate, sw_up, sw_down):
    Bsz, S, _ = hidden_states.shape
    F = w_gate.shape[2]
    FS = sw_gate.shape[1]
    nt = N // T

    x = hidden_states.reshape(N, D)
    x16 = x.astype(jnp.bfloat16)
    gwt = gate_weight.T
    wg16 = w_gate.astype(jnp.bfloat16)
    wu16 = w_up.astype(jnp.bfloat16)
    wd16 = w_down.astype(jnp.bfloat16)
    swg16 = sw_gate.astype(jnp.bfloat16)
    swu16 = sw_up.astype(jnp.bfloat16)
    swd16 = sw_down.astype(jnp.bfloat16)

    idx2d, w2d = pl.pallas_call(
        _router_body,
        grid=(nt,),
        in_specs=[
            pl.BlockSpec((T, D), lambda t: (t, 0)),
            pl.BlockSpec((D, E), lambda t: (0, 0)),
        ],
        out_specs=[
            pl.BlockSpec((T, E), lambda t: (t, 0)),
            pl.BlockSpec((T, E), lambda t: (t, 0)),
        ],
        out_shape=[
            jax.ShapeDtypeStruct((N, E), jnp.int32),
            jax.ShapeDtypeStruct((N, E), jnp.float32),
        ],
    )(x, gwt)

    sorted_tok, pairdst, slots = pl.kernel(
        _meta_body,
        out_type=[
            jax.ShapeDtypeStruct((NROWS,), jnp.int32),
            jax.ShapeDtypeStruct((NROWS,), jnp.int32),
            jax.ShapeDtypeStruct((32,), jnp.int32),
        ],
        mesh=plsc.VectorSubcoreMesh(
            core_axis_name="c", subcore_axis_name="s", num_cores=1),
        compiler_params=pltpu.CompilerParams(needs_layout_passes=False),
        scratch_types=[
            pltpu.VMEM((N,), jnp.int32),
            pltpu.VMEM((N,), jnp.int32),
            pltpu.VMEM((SEG_SZ,), jnp.int32),
            pltpu.VMEM((SEG_SZ,), jnp.int32),
            pltpu.VMEM((256,), jnp.int32),
            pltpu.VMEM((32,), jnp.int32),
        ],
    )(idx2d[:, 0], idx2d[:, 1])

    x_sorted = pl.kernel(
        _xgather_body,
        out_type=jax.ShapeDtypeStruct((NROWS, D), jnp.float32),
        mesh=plsc.VectorSubcoreMesh(core_axis_name="c", subcore_axis_name="s"),
        compiler_params=pltpu.CompilerParams(needs_layout_passes=False),
        scratch_types=[
            pltpu.VMEM((64,), jnp.int32),
            pltpu.VMEM((64, D), jnp.float32),
            pltpu.SemaphoreType.DMA,
        ],
    )(x, sorted_tok)

    out_buf = pl.pallas_call(
        _expert_body,
        grid_spec=pltpu.PrefetchScalarGridSpec(
            num_scalar_prefetch=1,
            grid=(NSLOT,),
            in_specs=[
                pl.BlockSpec((T, D), lambda t, m: (t, 0)),
                pl.BlockSpec((1, D, F), lambda t, m: (m[t], 0, 0)),
                pl.BlockSpec((1, D, F), lambda t, m: (m[t], 0, 0)),
                pl.BlockSpec((1, F, D), lambda t, m: (m[t], 0, 0)),
            ],
            out_specs=pl.BlockSpec((T, D), lambda t, m: (t, 0)),
        ),
        out_shape=jax.ShapeDtypeStruct((NROWS, D), jnp.float32),
    )(slots, x_sorted, wg16, wu16, wd16)

    ypairs = pl.kernel(
        _oscatter_body,
        out_type=jax.ShapeDtypeStruct((NPAIR_PAD, D), jnp.float32),
        mesh=plsc.VectorSubcoreMesh(core_axis_name="c", subcore_axis_name="s"),
        compiler_params=pltpu.CompilerParams(needs_layout_passes=False),
        scratch_types=[
            pltpu.VMEM((64,), jnp.int32),
            pltpu.VMEM((64, D), jnp.float32),
            pltpu.SemaphoreType.DMA,
        ],
    )(out_buf, pairdst)

    y = pl.pallas_call(
        _combine_body,
        grid=(nt,),
        in_specs=[
            pl.BlockSpec((T, D), lambda t: (t, 0)),
            pl.BlockSpec((T, D), lambda t: (t, 0)),
            pl.BlockSpec((T, D), lambda t: (t + N // T, 0)),
            pl.BlockSpec((T, E), lambda t: (t, 0)),
            pl.BlockSpec((D, FS), lambda t: (0, 0)),
            pl.BlockSpec((D, FS), lambda t: (0, 0)),
            pl.BlockSpec((FS, D), lambda t: (0, 0)),
        ],
        out_specs=pl.BlockSpec((T, D), lambda t: (t, 0)),
        out_shape=jax.ShapeDtypeStruct((N, D), jnp.float32),
    )(x16, ypairs, ypairs, w2d, swg16, swu16, swd16)

    return y.reshape(Bsz, S, D)
